# baseline (speedup 1.0000x reference)
import functools

import jax
import jax.numpy as jnp
from jax import lax
from jax.experimental import pallas as pl
from jax.experimental.pallas import tpu as pltpu
from jax.experimental.pallas import tpu_sc as plsc

_NUM_CORES = 2
_NUM_SUBCORES = 16
_NW = _NUM_CORES * _NUM_SUBCORES
_CHUNK = 128


@functools.partial(jax.jit, static_argnums=(2, 3, 4))
def _sc_gather(ids, table, n_chunks, chunk, dim):
    n = _NW * n_chunks * chunk
    mesh = plsc.VectorSubcoreMesh(core_axis_name="c", subcore_axis_name="s")

    @functools.partial(
        pl.kernel,
        mesh=mesh,
        out_type=jax.ShapeDtypeStruct((n, dim), jnp.float32),
        scratch_types=[
            pltpu.VMEM((n_chunks, chunk), jnp.int32),
            pltpu.VMEM((chunk, dim), jnp.float32),
            pltpu.SemaphoreType.DMA,
        ],
        compiler_params=pltpu.CompilerParams(use_tc_tiling_on_sc=False),
    )
    def k(ids_hbm, table_hbm, out_hbm, idx_v, rows_v, sem):
        cid = lax.axis_index("c")
        sid = lax.axis_index("s")
        wid = sid * _NUM_CORES + cid
        pltpu.sync_copy(ids_hbm.at[wid], idx_v)
        base = wid * (n_chunks * chunk)

        def body(j, carry):
            pltpu.async_copy(table_hbm.at[idx_v.at[j]], rows_v, sem).wait()
            pltpu.sync_copy(rows_v, out_hbm.at[pl.ds(base + j * chunk, chunk)])
            return carry

        lax.fori_loop(0, n_chunks, body, 0)

    return k(ids, table)


def kernel(input_ids, table):
    b, s = input_ids.shape
    _, dim = table.shape
    n = b * s
    assert n % (_NW * _CHUNK) == 0
    n_chunks = n // (_NW * _CHUNK)
    ids = input_ids.reshape(_NW, n_chunks, _CHUNK)
    out = _sc_gather(ids, table, n_chunks, _CHUNK, dim)
    return out.reshape(b, s, dim)


# speedup vs baseline: 1.1158x; 1.1158x over previous
import functools

import jax
import jax.numpy as jnp
from jax import lax
from jax.experimental import pallas as pl
from jax.experimental.pallas import tpu as pltpu
from jax.experimental.pallas import tpu_sc as plsc

_NUM_CORES = 2
_NUM_SUBCORES = 16
_NW = _NUM_CORES * _NUM_SUBCORES
_CHUNK = 128
_NBUF = 8
_LOOK = 5


@functools.partial(jax.jit, static_argnums=(2, 3, 4))
def _sc_gather(ids, table, n_chunks, chunk, dim):
    n = _NW * n_chunks * chunk
    assert n_chunks % _NBUF == 0 and n_chunks > 2 * _NBUF
    mesh = plsc.VectorSubcoreMesh(core_axis_name="c", subcore_axis_name="s")

    @functools.partial(
        pl.kernel,
        mesh=mesh,
        out_type=jax.ShapeDtypeStruct((n, dim), jnp.float32),
        scratch_types=[
            pltpu.VMEM((n_chunks, chunk), jnp.int32),
            pltpu.VMEM((_NBUF, chunk, dim), jnp.float32),
            pltpu.SemaphoreType.DMA((_NBUF,)),
            pltpu.SemaphoreType.DMA((_NBUF,)),
        ],
        compiler_params=pltpu.CompilerParams(use_tc_tiling_on_sc=False),
    )
    def k(ids_hbm, table_hbm, out_hbm, idx_v, rows_v, gsem, wsem):
        cid = lax.axis_index("c")
        sid = lax.axis_index("s")
        wid = sid * _NUM_CORES + cid
        pltpu.sync_copy(ids_hbm.at[wid], idx_v)
        base = wid * (n_chunks * chunk)

        def fire_gather(j, b):
            pltpu.async_copy(table_hbm.at[idx_v.at[j]], rows_v.at[b],
                             gsem.at[b])

        def wait_gather(b):
            pltpu.make_async_copy(table_hbm.at[idx_v.at[0]], rows_v.at[b],
                                  gsem.at[b]).wait()

        def fire_wb(j, b):
            pltpu.async_copy(rows_v.at[b], out_hbm.at[pl.ds(base + j * chunk,
                                                            chunk)],
                             wsem.at[b])

        def wait_wb(b):
            pltpu.make_async_copy(rows_v.at[b],
                                  out_hbm.at[pl.ds(base, chunk)],
                                  wsem.at[b]).wait()

        for j in range(_LOOK):
            fire_gather(j, j)
        for j in range(_NBUF - _LOOK):
            wait_gather(j % _NBUF)
            fire_wb(j, j % _NBUF)
            fire_gather(j + _LOOK, (j + _LOOK) % _NBUF)

        def body(j, carry):
            b = j % _NBUF
            bn = (j + _LOOK) % _NBUF
            wait_gather(b)
            fire_wb(j, b)
            wait_wb(bn)
            fire_gather(j + _LOOK, bn)
            return carry

        lax.fori_loop(_NBUF - _LOOK, n_chunks - _LOOK, body, 0)

        for j in range(n_chunks - _LOOK, n_chunks):
            b = j % _NBUF
            wait_gather(b)
            fire_wb(j, b)
        for b in range(_NBUF):
            wait_wb(b)

    return k(ids, table)


def kernel(input_ids, table):
    b, s = input_ids.shape
    _, dim = table.shape
    n = b * s
    assert n % (_NW * _CHUNK) == 0
    n_chunks = n // (_NW * _CHUNK)
    ids = input_ids.reshape(_NW, n_chunks, _CHUNK)
    out = _sc_gather(ids, table, n_chunks, _CHUNK, dim)
    return out.reshape(b, s, dim)


# speedup vs baseline: 3.6561x; 3.2767x over previous
import functools

import jax
import jax.numpy as jnp
from jax import lax
from jax.experimental import pallas as pl
from jax.experimental.pallas import tpu as pltpu
from jax.experimental.pallas import tpu_sc as plsc

_NUM_CORES = 2
_NUM_SUBCORES = 16
_NW = _NUM_CORES * _NUM_SUBCORES
_L = 16


_TBLK = 32768


def _tc_tr_body(x_ref, o_ref):
    xt = x_ref[...].T
    h = _TBLK // 2
    o_ref[...] = jnp.concatenate([xt[:h], xt[h:]], axis=1)


def _tc_detile(table_t):
    dim, vocab = table_t.shape
    grid = pl.cdiv(vocab, _TBLK)
    return pl.pallas_call(
        _tc_tr_body,
        grid=(grid,),
        in_specs=[pl.BlockSpec((dim, _TBLK), lambda i: (0, i))],
        out_specs=pl.BlockSpec((_TBLK // 2, 128), lambda i: (i, 0)),
        out_shape=jax.ShapeDtypeStruct((grid * _TBLK // 2, 128), jnp.float32),
    )(table_t)


@functools.partial(jax.jit, static_argnums=(2, 3))
def _sc_gather(ids, table_lin, seq, dim):
    nb = ids.shape[0]
    mesh = plsc.VectorSubcoreMesh(core_axis_name="c", subcore_axis_name="s")
    rt = dim // 8

    @functools.partial(
        pl.kernel,
        mesh=mesh,
        out_type=jax.ShapeDtypeStruct((seq, rt, nb, 8, 128), jnp.float32),
        scratch_types=[
            pltpu.VMEM((seq, 128), jnp.int32),
            pltpu.VMEM((3, 128, dim), jnp.float32),
            pltpu.VMEM((2, rt, 8, 129), jnp.float32),
            pltpu.SemaphoreType.DMA((3,)),
            pltpu.SemaphoreType.DMA((2,)),
        ],
        compiler_params=pltpu.CompilerParams(use_tc_tiling_on_sc=False,
                                             needs_layout_passes=False),
    )
    def k(ids_hbm, tab_hbm, out_hbm, idx_v, rows_v, tr_v, gsem, wsem):
        cid = lax.axis_index("c")
        sid = lax.axis_index("s")
        wid = sid * _NUM_CORES + cid
        pltpu.sync_copy(ids_hbm.at[wid], idx_v)

        def fire_g(s, b):
            pltpu.async_copy(tab_hbm.at[idx_v.at[s]], rows_v.at[b],
                             gsem.at[b])

        def wait_g(b):
            pltpu.make_async_copy(tab_hbm.at[idx_v.at[0]], rows_v.at[b],
                                  gsem.at[b]).wait()

        def fire_wb(s, b):
            pltpu.async_copy(tr_v.at[b, :, :, pl.ds(0, 128)],
                             out_hbm.at[s, :, wid], wsem.at[b])

        def wait_wb(b):
            pltpu.make_async_copy(tr_v.at[b, :, :, pl.ds(0, 128)],
                                  out_hbm.at[0, :, 0], wsem.at[b]).wait()

        lane = lax.iota(jnp.int32, _L)
        nl = dim // _L
        d_vecs = [lane + (l * _L) for l in range(nl)]
        r_vecs = [dv // 8 for dv in d_vecs]
        dl_vecs = [dv % 8 for dv in d_vecs]

        def transpose(rb, tb):
            @plsc.parallel_loop(0, 128, unroll=4)
            def blloop(bl):
                blv = jnp.full((_L,), 0, jnp.int32) + bl
                for l in range(nl):
                    x = rows_v[rb, bl, pl.ds(l * _L, _L)]
                    plsc.store_scatter(tr_v.at[tb],
                                       [r_vecs[l], dl_vecs[l], blv], x)

        fire_g(0, 0)
        fire_g(1, 1)
        for s in range(2):
            wait_g(s)
            fire_g(s + 2, (s + 2) % 3)
            transpose(s, s)
            fire_wb(s, s)

        def body(s, carry):
            rb = lax.rem(s, 3)
            tb = lax.rem(s, 2)
            wait_g(rb)
            fire_g(s + 2, lax.rem(s + 2, 3))
            wait_wb(tb)
            transpose(rb, tb)
            fire_wb(s, tb)
            return carry

        lax.fori_loop(2, seq - 2, body, 0)

        for s in range(seq - 2, seq):
            rb = s % 3
            tb = s % 2
            wait_g(rb)
            wait_wb(tb)
            transpose(rb, tb)
            fire_wb(s, tb)
        wait_wb(0)
        wait_wb(1)

    return k(ids, table_lin)


def kernel(input_ids, table):
    bsz, seq = input_ids.shape
    _, dim = table.shape
    nb = bsz // 128
    assert nb == _NW
    ids3 = input_ids.T.reshape(seq, nb, 128).transpose(1, 0, 2)
    half = _TBLK // 2
    ids3 = ((ids3 & ~(_TBLK - 1)) + ((ids3 & (half - 1)) * 2)
            + ((ids3 // half) & 1))
    tab = _tc_detile(table.T)
    tab_lin = tab.reshape(tab.shape[0] * 2, dim)
    out5 = _sc_gather(ids3, tab_lin, seq, dim)
    return jnp.transpose(out5, (2, 4, 0, 1, 3)).reshape(bsz, seq, dim)
